# Initial kernel scaffold; baseline (speedup 1.0000x reference)
import jax
import jax.numpy as jnp
from jax.experimental import pallas as pl


def kernel(x, edge_index, edge_attr, batch, params):
    raise NotImplementedError("write your pallas kernel here")


# speedup vs baseline: 5.9809x; 5.9809x over previous
import functools

import jax
import jax.numpy as jnp
from jax import lax
from jax.experimental import pallas as pl
from jax.experimental.pallas import tpu as pltpu
from jax.experimental.pallas import tpu_sc as plsc

_N = 10000
_E = 320000
_NE = 16
_NINP = 128
_NOUT1 = 64
_NOUT2 = 16
_NIN = _NOUT1 + _NOUT2
_G = 128

_NC = 2
_NS = 16
_NW = _NC * _NS
_EPW = _E // _NW
_CH = 40
_NCHUNK = _EPW // _CH
_NPAD = 10240
_RPT = _NPAD // _NS
_ZC = _NE * _NOUT1


def _sc_layer_body(z_hbm, ea_hbm, src_hbm, dst_hbm, out_hbm,
                   idx_v, didx_v, ea_v, rows_v, msg_v, sem):
    c = lax.axis_index("c")
    s = lax.axis_index("s")
    wid = s * _NC + c
    base0 = wid * _EPW

    def chunk_body(j, carry):
        base = base0 + j * _CH
        pltpu.sync_copy(src_hbm.at[pl.ds(base, _CH)], idx_v)
        pltpu.sync_copy(dst_hbm.at[pl.ds(base, _CH)], didx_v)
        pltpu.sync_copy(ea_hbm.at[pl.ds(base, _CH)], ea_v)
        pltpu.async_copy(z_hbm.at[idx_v], rows_v, sem).wait()

        def edge_body(e, carry2):
            accs = [jnp.zeros((16,), jnp.float32) for _ in range(4)]
            eav = ea_v[e, :]
            for i in range(_NE):
                a = eav[i]
                for ob in range(4):
                    accs[ob] = accs[ob] + a * rows_v[e, pl.ds(i * 64 + ob * 16, 16)]
            for ob in range(4):
                msg_v[e, pl.ds(ob * 16, 16)] = accs[ob]
            return carry2

        lax.fori_loop(0, _CH, edge_body, 0)
        pltpu.sync_copy(msg_v, out_hbm.at[pl.ds(base, _CH)])
        return carry

    lax.fori_loop(0, _NCHUNK, chunk_body, 0)


def _sc_layer(z, ea, src, dst):
    mesh = plsc.VectorSubcoreMesh(core_axis_name="c", subcore_axis_name="s")
    call = pl.kernel(
        _sc_layer_body,
        mesh=mesh,
        out_type=jax.ShapeDtypeStruct((_E, _NOUT1), jnp.float32),
        scratch_types=[
            pltpu.VMEM((_CH,), jnp.int32),
            pltpu.VMEM((_CH,), jnp.int32),
            pltpu.VMEM((_CH, _NE), jnp.float32),
            pltpu.VMEM((_CH, _ZC), jnp.float32),
            pltpu.VMEM((_CH, _NOUT1), jnp.float32),
            pltpu.SemaphoreType.DMA,
        ],
    )
    return call(z, ea, src, dst)


_EB = 2000
_NB = 1000
_PB = 200


def _edge_mlp_body(eattr, f1, f2, f3, w4, o1, o2, o3, o4):
    ea = eattr[...]
    e1 = jnp.maximum(jnp.dot(ea, f1[...], preferred_element_type=jnp.float32), 0.0)
    e2 = (jnp.tanh(jnp.dot(ea, f2[...], preferred_element_type=jnp.float32)) *
          jnp.tanh(jnp.dot(ea, f3[...], preferred_element_type=jnp.float32)))
    parts = []
    for l in range(4):
        parts.append(e1[:, l * 32:(l + 1) * 32])
        parts.append(e2[:, l * 32:(l + 1) * 32])
    cc = jnp.concatenate(parts, axis=1)
    out = jnp.maximum(jnp.dot(cc, w4[...], preferred_element_type=jnp.float32), 0.0)
    o1[...] = out[:, 0:16]
    o2[...] = out[:, 16:32]
    o3[...] = out[:, 32:48]
    o4[...] = out[:, 48:64]


def _edge_mlp(edge_attr, f1, f2, f3, w4):
    nblk = _E // _EB
    out_sh = jax.ShapeDtypeStruct((_E, _NE), jnp.float32)
    wspec = lambda shape: pl.BlockSpec(shape, lambda i: (0, 0))
    return pl.pallas_call(
        _edge_mlp_body,
        grid=(nblk,),
        in_specs=[
            pl.BlockSpec((_EB, _NE), lambda i: (i, 0)),
            wspec(f1.shape), wspec(f2.shape), wspec(f3.shape), wspec(w4.shape),
        ],
        out_specs=[pl.BlockSpec((_EB, _NE), lambda i: (i, 0))] * 4,
        out_shape=[out_sh] * 4,
    )(edge_attr, f1, f2, f3, w4)


def _dense_pre_body(x, w2d, f11w, f11b, f12w, f12b, z, bp):
    h = x[...]
    z[...] = jnp.dot(h, w2d[...], preferred_element_type=jnp.float32)
    bp[...] = (jnp.tanh(jnp.dot(h, f11w[...], preferred_element_type=jnp.float32) + f11b[...]) *
               jnp.tanh(jnp.dot(h, f12w[...], preferred_element_type=jnp.float32) + f12b[...]))


def _dense_pre(x, w2d, f11w, f11b, f12w, f12b):
    nblk = _N // _NB
    wspec = lambda a: pl.BlockSpec(a.shape, lambda i: tuple(0 for _ in a.shape))
    return pl.pallas_call(
        _dense_pre_body,
        grid=(nblk,),
        in_specs=[
            pl.BlockSpec((_NB, x.shape[1]), lambda i: (i, 0)),
            wspec(w2d), wspec(f11w), wspec(f11b), wspec(f12w), wspec(f12b),
        ],
        out_specs=[pl.BlockSpec((_NB, _ZC), lambda i: (i, 0)),
                   pl.BlockSpec((_NB, _NOUT2), lambda i: (i, 0))],
        out_shape=[jax.ShapeDtypeStruct((_N, _ZC), jnp.float32),
                   jax.ShapeDtypeStruct((_N, _NOUT2), jnp.float32)],
    )(x, w2d, f11w, f11b, f12w, f12b)


def _dense_mid_body(acc, bprev, cb, w2d, f11w, f11b, f12w, f12b, z, bp, h_out):
    a = jnp.maximum(acc[...] + cb[...], 0.0)
    h = jnp.concatenate([a, bprev[...]], axis=1)
    h_out[...] = h
    z[...] = jnp.dot(h, w2d[...], preferred_element_type=jnp.float32)
    bp[...] = (jnp.tanh(jnp.dot(h, f11w[...], preferred_element_type=jnp.float32) + f11b[...]) *
               jnp.tanh(jnp.dot(h, f12w[...], preferred_element_type=jnp.float32) + f12b[...]))


def _dense_mid(acc, bprev, cb, w2d, f11w, f11b, f12w, f12b):
    nblk = _N // _NB
    wspec = lambda a: pl.BlockSpec(a.shape, lambda i: tuple(0 for _ in a.shape))
    return pl.pallas_call(
        _dense_mid_body,
        grid=(nblk,),
        in_specs=[
            pl.BlockSpec((_NB, _NOUT1), lambda i: (i, 0)),
            pl.BlockSpec((_NB, _NOUT2), lambda i: (i, 0)),
            wspec(cb), wspec(w2d), wspec(f11w), wspec(f11b), wspec(f12w), wspec(f12b),
        ],
        out_specs=[pl.BlockSpec((_NB, _ZC), lambda i: (i, 0)),
                   pl.BlockSpec((_NB, _NOUT2), lambda i: (i, 0)),
                   pl.BlockSpec((_NB, _NIN), lambda i: (i, 0))],
        out_shape=[jax.ShapeDtypeStruct((_N, _ZC), jnp.float32),
                   jax.ShapeDtypeStruct((_N, _NOUT2), jnp.float32),
                   jax.ShapeDtypeStruct((_N, _NIN), jnp.float32)],
    )(acc, bprev, cb, w2d, f11w, f11b, f12w, f12b)


def _dense_fin_body(acc, bprev, cb, h_out):
    a = jnp.maximum(acc[...] + cb[...], 0.0)
    h_out[...] = jnp.concatenate([a, bprev[...]], axis=1)


def _dense_fin(acc, bprev, cb):
    nblk = _N // _NB
    return pl.pallas_call(
        _dense_fin_body,
        grid=(nblk,),
        in_specs=[
            pl.BlockSpec((_NB, _NOUT1), lambda i: (i, 0)),
            pl.BlockSpec((_NB, _NOUT2), lambda i: (i, 0)),
            pl.BlockSpec(cb.shape, lambda i: (0, 0)),
        ],
        out_specs=pl.BlockSpec((_NB, _NIN), lambda i: (i, 0)),
        out_shape=jax.ShapeDtypeStruct((_N, _NIN), jnp.float32),
    )(acc, bprev, cb)


def _pool_body(h_ref, b_ref, f1w, f1b, f2w, f2b, o_ref, sum_acc, max_acc, cnt_acc):
    i = pl.program_id(0)

    @pl.when(i == 0)
    def _init():
        sum_acc[...] = jnp.zeros_like(sum_acc)
        max_acc[...] = jnp.full_like(max_acc, -1e30)
        cnt_acc[...] = jnp.zeros_like(cnt_acc)

    b = b_ref[0, 0, :]
    h = h_ref[...]
    gids = lax.broadcasted_iota(jnp.int32, (_G, _PB), 0)
    mask = gids == b[None, :]
    mf = mask.astype(jnp.float32)
    sum_acc[...] += jnp.dot(mf, h, preferred_element_type=jnp.float32)
    cnt_acc[...] += jnp.sum(mf, axis=1, keepdims=True)
    big = h[None, :, :] + (mf[:, :, None] - 1.0) * 1e30
    max_acc[...] = jnp.maximum(max_acc[...], jnp.max(big, axis=1))

    @pl.when(i == pl.num_programs(0) - 1)
    def _fin():
        cnt = jnp.maximum(cnt_acc[...], 1.0)
        mean = sum_acc[...] / cnt
        mx = max_acc[...]
        mx = jnp.where(mx < -1e29, 0.0, mx)
        g = jnp.concatenate([mean, mx], axis=1)
        t = jnp.maximum(jnp.dot(g, f1w[...], preferred_element_type=jnp.float32) + f1b[...], 0.0)
        logits = jnp.dot(t, f2w[...], preferred_element_type=jnp.float32) + f2b[...]
        m = jnp.max(logits, axis=1, keepdims=True)
        lse = jnp.log(jnp.sum(jnp.exp(logits - m), axis=1, keepdims=True)) + m
        o_ref[...] = logits - lse


def _pool(h, batch3, f1w, f1b, f2w, f2b):
    nblk = _N // _PB
    wspec = lambda a: pl.BlockSpec(a.shape, lambda i: tuple(0 for _ in a.shape))
    return pl.pallas_call(
        _pool_body,
        grid=(nblk,),
        in_specs=[
            pl.BlockSpec((_PB, _NIN), lambda i: (i, 0)),
            pl.BlockSpec((1, 1, _PB), lambda i: (i, 0, 0)),
            wspec(f1w), wspec(f1b), wspec(f2w), wspec(f2b),
        ],
        out_specs=pl.BlockSpec((_G, 2), lambda i: (0, 0)),
        out_shape=jax.ShapeDtypeStruct((_G, 2), jnp.float32),
        scratch_shapes=[
            pltpu.VMEM((_G, _NIN), jnp.float32),
            pltpu.VMEM((_G, _NIN), jnp.float32),
            pltpu.VMEM((_G, 1), jnp.float32),
        ],
    )(h, batch3, f1w, f1b, f2w, f2b)


def kernel(x, edge_index, edge_attr, batch, params):
    src = jnp.asarray(edge_index[0])
    dst = jnp.asarray(edge_index[1])

    layers = [params['conv1'], params['conv2'], params['conv3'], params['conv4']]
    f1 = jnp.concatenate([p['fc1_1'] for p in layers], axis=1)
    f2 = jnp.concatenate([p['fc1_2'] for p in layers], axis=1)
    f3 = jnp.concatenate([p['fc1_3'] for p in layers], axis=1)
    w4 = jnp.zeros((4 * 64, 64), jnp.float32)
    for l, p in enumerate(layers):
        w4 = w4.at[l * 64:(l + 1) * 64, l * 16:(l + 1) * 16].set(p['fc1_4'])
    w2d = [p['conv_w'].transpose(1, 0, 2).reshape(p['conv_w'].shape[1], _ZC)
           for p in layers]
    f11w = [p['fc11_w'] for p in layers]
    f11b = [p['fc11_b'].reshape(1, _NOUT2) for p in layers]
    f12w = [p['fc12_w'] for p in layers]
    f12b = [p['fc12_b'].reshape(1, _NOUT2) for p in layers]
    cb = [p['conv_b'].reshape(1, _NOUT1) for p in layers]

    ea1, ea2, ea3, ea4 = _edge_mlp(edge_attr, f1, f2, f3, w4)
    eas = [ea1, ea2, ea3, ea4]

    z, bp = _dense_pre(x, w2d[0], f11w[0], f11b[0], f12w[0], f12b[0])
    for l in range(3):
        msg = _sc_layer(z, eas[l], src, dst)
        acc = jax.ops.segment_sum(msg, dst, num_segments=_N)
        z, bp, _h = _dense_mid(acc, bp, cb[l], w2d[l + 1],
                               f11w[l + 1], f11b[l + 1], f12w[l + 1], f12b[l + 1])
    msg = _sc_layer(z, eas[3], src, dst)
    acc = jax.ops.segment_sum(msg, dst, num_segments=_N)
    h4 = _dense_fin(acc, bp, cb[3])

    batch3 = batch.reshape(_N // _PB, 1, _PB)
    return _pool(h4, batch3, params['fc1_w'], params['fc1_b'].reshape(1, 100),
                 params['fc2_w'], params['fc2_b'].reshape(1, 2))
